# Initial kernel scaffold; baseline (speedup 1.0000x reference)
import jax
import jax.numpy as jnp
from jax.experimental import pallas as pl


def kernel(x, global_pe, week_pe, month_pe, year_pe):
    raise NotImplementedError("write your pallas kernel here")


# speedup vs baseline: 1.9831x; 1.9831x over previous
import jax
import jax.numpy as jnp
from jax.experimental import pallas as pl
from jax.experimental.pallas import tpu as pltpu

_BS = 512


def _cyclic_build_body(week_ref, month_ref, year_ref, out_ref):
    s = out_ref.shape[0]
    d_part = week_ref.shape[1]
    pos = jax.lax.broadcasted_iota(jnp.int32, (s, 1), 0)

    def expand(tbl_ref):
        n = tbl_ref.shape[0]
        idx = pos % n
        onehot = (idx == jax.lax.broadcasted_iota(jnp.int32, (s, n), 1)).astype(jnp.float32)
        return jnp.dot(onehot, tbl_ref[...], preferred_element_type=jnp.float32)

    out_ref[:, 0:d_part] = expand(week_ref)
    out_ref[:, d_part:2 * d_part] = expand(month_ref)
    out_ref[:, 2 * d_part:3 * d_part] = expand(year_ref)


def _add_body(x_ref, g_ref, c_ref, out_ref):
    dg = g_ref.shape[1]
    out_ref[0, :, 0:dg] = x_ref[0, :, 0:dg] + g_ref[...]
    out_ref[0, :, dg:] = x_ref[0, :, dg:] + c_ref[...]


def kernel(x, global_pe, week_pe, month_pe, year_pe):
    b, s, d = x.shape
    d_part = week_pe.shape[1]
    d_global = d - 3 * d_part

    cyclic = pl.pallas_call(
        _cyclic_build_body,
        out_shape=jax.ShapeDtypeStruct((s, 3 * d_part), jnp.float32),
    )(week_pe, month_pe, year_pe)

    out = pl.pallas_call(
        _add_body,
        out_shape=jax.ShapeDtypeStruct((b, s, d), jnp.float32),
        grid=(s // _BS, b),
        in_specs=[
            pl.BlockSpec((1, _BS, d), lambda i, j: (j, i, 0)),
            pl.BlockSpec((_BS, d_global), lambda i, j: (i, 0)),
            pl.BlockSpec((_BS, 3 * d_part), lambda i, j: (i, 0)),
        ],
        out_specs=pl.BlockSpec((1, _BS, d), lambda i, j: (j, i, 0)),
        compiler_params=pltpu.CompilerParams(
            dimension_semantics=("arbitrary", "arbitrary"),
        ),
    )(x, global_pe, cyclic)
    return out
